# Initial kernel scaffold; baseline (speedup 1.0000x reference)
import jax
import jax.numpy as jnp
from jax.experimental import pallas as pl


def kernel(x, edge_attr, params, edge_index, batch):
    raise NotImplementedError("write your pallas kernel here")


# speedup vs baseline: 2.0043x; 2.0043x over previous
import functools

import jax
import jax.numpy as jnp
from jax import lax
from jax.experimental import pallas as pl
from jax.experimental.pallas import tpu as pltpu
from jax.experimental.pallas import tpu_sc as plsc

_N = 10000
_E = 320000
_D = 128
_H = 128
_DE = 16
_NLAYERS = 5
_NG = 256

_NC = 2
_NS = 16
_NW = _NC * _NS
_J = 128
_KC = 16
_K = 80
_EW = _K * _J
_EP = _NW * _EW
_AN = 10112
_RPT = _AN // _NS
_RCH = (128, 128, 128, 128, 120)


def _sc_edge_body(h_hbm, ea_hbm, src_hbm, dst_hbm, out_hbm,
                  idx_s, idx_d, gbuf, ebuf, gsem, esem, aggr):
    cid = lax.axis_index("c")
    sid = lax.axis_index("s")
    wid = sid * _NC + cid

    zero16 = jnp.zeros((16,), jnp.float32)

    def _zrow(i, c):
        for cc in range(_H // 16):
            gbuf[i, pl.ds(cc * 16, 16)] = zero16
        return c

    lax.fori_loop(0, _J, _zrow, 0)
    base = sid * _RPT
    off = 0
    for rc in _RCH:
        pltpu.sync_copy(gbuf.at[pl.ds(0, rc)], aggr.at[pl.ds(base + off, rc)])
        off += rc
    plsc.subcore_barrier()

    ebase = wid * _EW

    def _chunk(jc, c0):
        pltpu.sync_copy(src_hbm.at[wid, pl.ds(jc * _KC, _KC)], idx_s)
        pltpu.sync_copy(dst_hbm.at[wid, pl.ds(jc * _KC, _KC)], idx_d)

        def _body(j, c):
            g = pltpu.async_copy(h_hbm.at[idx_s.at[j]], gbuf, gsem)
            e = pltpu.async_copy(
                ea_hbm.at[pl.ds(ebase + (jc * _KC + j) * _J, _J)], ebuf, esem)
            g.wait()
            e.wait()

            def _crow(i, c2):
                for cc in range(_H // 16):
                    sl = pl.ds(cc * 16, 16)
                    gbuf[i, sl] = jnp.maximum(gbuf[i, sl] + ebuf[i, sl], 0.0)
                return c2

            lax.fori_loop(0, _J, _crow, 0)
            pltpu.sync_copy(gbuf, aggr.at[idx_d.at[j]], add=True)
            return c

        return lax.fori_loop(0, _KC, _body, c0)

    lax.fori_loop(0, _K // _KC, _chunk, 0)

    plsc.subcore_barrier()
    off = 0
    for rc in _RCH:
        sl = pl.ds(base + off, rc)
        pltpu.sync_copy(aggr.at[sl], out_hbm.at[cid, sl])
        off += rc


_sc_edge = pl.kernel(
    _sc_edge_body,
    out_type=jax.ShapeDtypeStruct((_NC, _AN, _H), jnp.float32),
    mesh=plsc.VectorSubcoreMesh(core_axis_name="c", subcore_axis_name="s",
                                num_cores=_NC, num_subcores=_NS),
    scratch_types=[
        pltpu.VMEM((_KC, _J), jnp.int32),
        pltpu.VMEM((_KC, _J), jnp.int32),
        pltpu.VMEM((_J, _H), jnp.float32),
        pltpu.VMEM((_J, _H), jnp.float32),
        pltpu.SemaphoreType.DMA,
        pltpu.SemaphoreType.DMA,
        pltpu.VMEM_SHARED((_AN, _H), jnp.float32),
    ],
)


def _ea_body(attr, wb, bb, out):
    out[...] = jnp.dot(attr[...], wb[...],
                       preferred_element_type=jnp.float32) + bb[...]


_BE = 4096
_ea_call = pl.pallas_call(
    _ea_body,
    grid=(_EP // _BE,),
    in_specs=[
        pl.BlockSpec((_BE, _DE), lambda i: (i, 0)),
        pl.BlockSpec((_DE, _H), lambda i: (0, 0)),
        pl.BlockSpec((1, _H), lambda i: (0, 0)),
    ],
    out_specs=pl.BlockSpec((_BE, _H), lambda i: (i, 0)),
    out_shape=jax.ShapeDtypeStruct((_EP, _H), jnp.float32),
)


def _dense_body(h, agg, w1, b1, g1, be1, w2, b2, g2, be2, wfc, s_in,
                h_out, s_out):
    z = h[...] + agg[0, :_N, :] + agg[1, :_N, :]
    a = jnp.dot(z, w1[...], preferred_element_type=jnp.float32) + b1[...]
    m = jnp.mean(a, axis=0, keepdims=True)
    v = jnp.mean((a - m) ** 2, axis=0, keepdims=True)
    a = jnp.maximum((a - m) / jnp.sqrt(v + 1e-5) * g1[...] + be1[...], 0.0)
    zz = jnp.dot(a, w2[...], preferred_element_type=jnp.float32) + b2[...]
    m2 = jnp.mean(zz, axis=0, keepdims=True)
    v2 = jnp.mean((zz - m2) ** 2, axis=0, keepdims=True)
    hh = jnp.maximum((zz - m2) / jnp.sqrt(v2 + 1e-5) * g2[...] + be2[...], 0.0)
    h_out[...] = hh
    s_out[...] = s_in[...] + jnp.dot(hh, wfc[...],
                                     preferred_element_type=jnp.float32)


_dense_call = pl.pallas_call(
    _dense_body,
    out_shape=(
        jax.ShapeDtypeStruct((_N, _H), jnp.float32),
        jax.ShapeDtypeStruct((_N, _H), jnp.float32),
    ),
)


def _pool_body(s, x, wfc0, bstack, batch, out):
    total = s[...] + jnp.dot(x[...], wfc0[...],
                             preferred_element_type=jnp.float32)
    gid = lax.broadcasted_iota(jnp.int32, (_NG, _N), 0)
    p = jnp.where(gid == batch[...], 1.0, 0.0)
    pooled = jnp.dot(p, total, preferred_element_type=jnp.float32)
    counts = jnp.sum(p, axis=1, keepdims=True)
    out[...] = pooled / jnp.maximum(counts, 1.0) + jnp.sum(
        bstack[...], axis=0, keepdims=True)


_pool_call = pl.pallas_call(
    _pool_body,
    out_shape=jax.ShapeDtypeStruct((_NG, _H), jnp.float32),
)


def kernel(x, edge_attr, params, edge_index, batch):
    src = edge_index[0].astype(jnp.int32)
    dst = edge_index[1].astype(jnp.int32)
    src = jnp.pad(src, (0, _EP - _E)).reshape(_NW, _K, _J)
    dst = jnp.pad(dst, (0, _EP - _E), constant_values=_N).reshape(_NW, _K, _J)
    ea_pad = jnp.pad(edge_attr.astype(jnp.float32), ((0, _EP - _E), (0, 0)))
    b2d = batch.astype(jnp.int32).reshape(1, _N)

    h = x.astype(jnp.float32)
    s = jnp.zeros((_N, _H), jnp.float32)
    for i in range(_NLAYERS):
        p = params["layers"][i]
        ea = _ea_call(ea_pad, p["Wb"], p["bb"].reshape(1, _H))
        agg = _sc_edge(h, ea, src, dst)
        h, s = _dense_call(
            h, agg, p["W1"], p["b1"].reshape(1, _H), p["g1"].reshape(1, _H),
            p["be1"].reshape(1, _H), p["W2"], p["b2"].reshape(1, _H),
            p["g_out"].reshape(1, _H), p["be_out"].reshape(1, _H),
            params["fcs"][i + 1][0], s)

    bstack = jnp.stack([b for (_w, b) in params["fcs"]])
    return _pool_call(s, x.astype(jnp.float32),
                      params["fcs"][0][0], bstack, b2d)


# speedup vs baseline: 2.3291x; 1.1620x over previous
import functools

import jax
import jax.numpy as jnp
from jax import lax
from jax.experimental import pallas as pl
from jax.experimental.pallas import tpu as pltpu
from jax.experimental.pallas import tpu_sc as plsc

_N = 10000
_E = 320000
_D = 128
_H = 128
_DE = 16
_NLAYERS = 5
_NG = 256

_NC = 2
_NS = 16
_NW = _NC * _NS
_J = 40
_KB = 256
_CH = 32
_NCH = _KB // _CH
_EW = _KB * _J
_EP = _NW * _EW
_AN = 10112
_RPT = _AN // _NS
_RCH = (128, 128, 128, 128, 120)


def _sc_edge_body(h_hbm, ea_hbm, src_hbm, dst_hbm, out_hbm,
                  is0, is1, id0, id1, g0, g1, e0, e1, s0, s1,
                  ism0, ism1, gs0, gs1, es0, es1, ss0, ss1, aggr):
    cid = lax.axis_index("c")
    sid = lax.axis_index("s")
    wid = sid * _NC + cid

    isb = (is0, is1)
    idb = (id0, id1)
    isem = (ism0, ism1)
    gbufs = (g0, g1)
    ebufs = (e0, e1)
    sbufs = (s0, s1)
    gsems = (gs0, gs1)
    esems = (es0, es1)
    ssems = (ss0, ss1)

    zero16 = jnp.zeros((16,), jnp.float32)

    def _zrow(i, c):
        for cc in range(_H // 16):
            g0[i, pl.ds(cc * 16, 16)] = zero16
        return c

    lax.fori_loop(0, _J, _zrow, 0)
    base = sid * _RPT
    for r in range(_RPT // _J):
        pltpu.sync_copy(g0, aggr.at[pl.ds(base + r * _J, _J)])
    rem = _RPT - (_RPT // _J) * _J
    if rem:
        pltpu.sync_copy(g0.at[pl.ds(0, rem)],
                        aggr.at[pl.ds(base + _RPT - rem, rem)])
    plsc.subcore_barrier()

    def _ea_ref(j):
        return ea_hbm.at[pl.ds(wid * _EW + j * _J, _J)]

    def _issue_ge(isrow, j, p):
        pltpu.async_copy(h_hbm.at[isrow], gbufs[p], gsems[p])
        pltpu.async_copy(_ea_ref(j), ebufs[p], esems[p])

    def _wait_ge(isrow, j, p):
        pltpu.make_async_copy(h_hbm.at[isrow], gbufs[p], gsems[p]).wait()
        pltpu.make_async_copy(_ea_ref(j), ebufs[p], esems[p]).wait()

    def _compute(p):
        gb, eb, sb = gbufs[p], ebufs[p], sbufs[p]

        @plsc.parallel_loop(0, _J)
        def _(i):
            for cc in range(_H // 16):
                sl = pl.ds(cc * 16, 16)
                sb[i, sl] = jnp.maximum(gb[i, sl] + eb[i, sl], 0.0)

    def _issue_scatter(idrow, p):
        pltpu.async_copy(sbufs[p], aggr.at[idrow], ssems[p], add=True)

    def _wait_scatter(idrow, p):
        pltpu.make_async_copy(sbufs[p], aggr.at[idrow], ssems[p]).wait()

    stage_descs = None
    for ch in range(_NCH):
        ip = ch % 2
        isv, idv = isb[ip], idb[ip]
        if ch == 0:
            pltpu.sync_copy(src_hbm.at[wid, pl.ds(0, _CH)], isv)
            pltpu.sync_copy(dst_hbm.at[wid, pl.ds(0, _CH)], idv)
        else:
            stage_descs[0].wait()
            stage_descs[1].wait()
        if ch < _NCH - 1:
            nxt = (ch + 1) % 2
            sl = pl.ds((ch + 1) * _CH, _CH)
            stage_descs = (
                pltpu.async_copy(src_hbm.at[wid, sl], isb[nxt], isem[nxt]),
                pltpu.async_copy(dst_hbm.at[wid, sl], idb[nxt], isem[nxt]),
            )
        jbase = ch * _CH
        _issue_ge(isv.at[0], jbase + 0, 0)
        _issue_ge(isv.at[1], jbase + 1, 1)
        for b in (0, 1):
            _wait_ge(isv.at[b], jbase + b, b)
            _compute(b)
            _issue_scatter(idv.at[b], b)
            _issue_ge(isv.at[b + 2], jbase + b + 2, b)

        def _pair(q, c):
            for p in (0, 1):
                b = 2 * q + p
                _wait_ge(isv.at[b], jbase + b, p)
                _wait_scatter(idv.at[b - 2], p)
                _compute(p)
                _issue_scatter(idv.at[b], p)
                _issue_ge(isv.at[b + 2], jbase + b + 2, p)
            return c

        lax.fori_loop(1, _CH // 2 - 1, _pair, 0)
        for p in (0, 1):
            b = _CH - 2 + p
            _wait_ge(isv.at[b], jbase + b, p)
            _wait_scatter(idv.at[b - 2], p)
            _compute(p)
            _issue_scatter(idv.at[b], p)
        for p in (0, 1):
            _wait_scatter(idv.at[_CH - 2 + p], p)

    plsc.subcore_barrier()
    off = 0
    for rc in _RCH:
        sl = pl.ds(base + off, rc)
        pltpu.sync_copy(aggr.at[sl], out_hbm.at[cid, sl])
        off += rc


_sc_edge = pl.kernel(
    _sc_edge_body,
    out_type=jax.ShapeDtypeStruct((_NC, _AN, _H), jnp.float32),
    mesh=plsc.VectorSubcoreMesh(core_axis_name="c", subcore_axis_name="s",
                                num_cores=_NC, num_subcores=_NS),
    scratch_types=[
        pltpu.VMEM((_CH, _J), jnp.int32),
        pltpu.VMEM((_CH, _J), jnp.int32),
        pltpu.VMEM((_CH, _J), jnp.int32),
        pltpu.VMEM((_CH, _J), jnp.int32),
        pltpu.VMEM((_J, _H), jnp.float32),
        pltpu.VMEM((_J, _H), jnp.float32),
        pltpu.VMEM((_J, _H), jnp.float32),
        pltpu.VMEM((_J, _H), jnp.float32),
        pltpu.VMEM((_J, _H), jnp.float32),
        pltpu.VMEM((_J, _H), jnp.float32),
        pltpu.SemaphoreType.DMA,
        pltpu.SemaphoreType.DMA,
        pltpu.SemaphoreType.DMA,
        pltpu.SemaphoreType.DMA,
        pltpu.SemaphoreType.DMA,
        pltpu.SemaphoreType.DMA,
        pltpu.SemaphoreType.DMA,
        pltpu.SemaphoreType.DMA,
        pltpu.VMEM_SHARED((_AN, _H), jnp.float32),
    ],
)


def _ea_body(attr, wb, bb, out):
    out[...] = jnp.dot(attr[...], wb[...],
                       preferred_element_type=jnp.float32) + bb[...]


_BE = 4096
_ea_call = pl.pallas_call(
    _ea_body,
    grid=(_EP // _BE,),
    in_specs=[
        pl.BlockSpec((_BE, _DE), lambda i: (i, 0)),
        pl.BlockSpec((_DE, _H), lambda i: (0, 0)),
        pl.BlockSpec((1, _H), lambda i: (0, 0)),
    ],
    out_specs=pl.BlockSpec((_BE, _H), lambda i: (i, 0)),
    out_shape=jax.ShapeDtypeStruct((_EP, _H), jnp.float32),
)


def _dense_body(h, agg, w1, b1, g1, be1, w2, b2, g2, be2, wfc, s_in,
                h_out, s_out):
    z = h[...] + agg[0, :_N, :] + agg[1, :_N, :]
    a = jnp.dot(z, w1[...], preferred_element_type=jnp.float32) + b1[...]
    m = jnp.mean(a, axis=0, keepdims=True)
    v = jnp.mean((a - m) ** 2, axis=0, keepdims=True)
    a = jnp.maximum((a - m) / jnp.sqrt(v + 1e-5) * g1[...] + be1[...], 0.0)
    zz = jnp.dot(a, w2[...], preferred_element_type=jnp.float32) + b2[...]
    m2 = jnp.mean(zz, axis=0, keepdims=True)
    v2 = jnp.mean((zz - m2) ** 2, axis=0, keepdims=True)
    hh = jnp.maximum((zz - m2) / jnp.sqrt(v2 + 1e-5) * g2[...] + be2[...], 0.0)
    h_out[...] = hh
    s_out[...] = s_in[...] + jnp.dot(hh, wfc[...],
                                     preferred_element_type=jnp.float32)


_dense_call = pl.pallas_call(
    _dense_body,
    out_shape=(
        jax.ShapeDtypeStruct((_N, _H), jnp.float32),
        jax.ShapeDtypeStruct((_N, _H), jnp.float32),
    ),
)


def _pool_body(s, x, wfc0, bstack, batch, out):
    total = s[...] + jnp.dot(x[...], wfc0[...],
                             preferred_element_type=jnp.float32)
    gid = lax.broadcasted_iota(jnp.int32, (_NG, _N), 0)
    p = jnp.where(gid == batch[...], 1.0, 0.0)
    pooled = jnp.dot(p, total, preferred_element_type=jnp.float32)
    counts = jnp.sum(p, axis=1, keepdims=True)
    out[...] = pooled / jnp.maximum(counts, 1.0) + jnp.sum(
        bstack[...], axis=0, keepdims=True)


_pool_call = pl.pallas_call(
    _pool_body,
    out_shape=jax.ShapeDtypeStruct((_NG, _H), jnp.float32),
)


def kernel(x, edge_attr, params, edge_index, batch):
    src = edge_index[0].astype(jnp.int32)
    dst = edge_index[1].astype(jnp.int32)
    src = jnp.pad(src, (0, _EP - _E)).reshape(_NW, _KB, _J)
    dst = jnp.pad(dst, (0, _EP - _E), constant_values=_N).reshape(_NW, _KB, _J)
    ea_pad = jnp.pad(edge_attr.astype(jnp.float32), ((0, _EP - _E), (0, 0)))
    b2d = batch.astype(jnp.int32).reshape(1, _N)

    h = x.astype(jnp.float32)
    s = jnp.zeros((_N, _H), jnp.float32)
    for i in range(_NLAYERS):
        p = params["layers"][i]
        ea = _ea_call(ea_pad, p["Wb"], p["bb"].reshape(1, _H))
        agg = _sc_edge(h, ea, src, dst)
        h, s = _dense_call(
            h, agg, p["W1"], p["b1"].reshape(1, _H), p["g1"].reshape(1, _H),
            p["be1"].reshape(1, _H), p["W2"], p["b2"].reshape(1, _H),
            p["g_out"].reshape(1, _H), p["be_out"].reshape(1, _H),
            params["fcs"][i + 1][0], s)

    bstack = jnp.stack([b for (_w, b) in params["fcs"]])
    return _pool_call(s, x.astype(jnp.float32),
                      params["fcs"][0][0], bstack, b2d)


# speedup vs baseline: 2.4300x; 1.0433x over previous
import functools

import jax
import jax.numpy as jnp
from jax import lax
from jax.experimental import pallas as pl
from jax.experimental.pallas import tpu as pltpu
from jax.experimental.pallas import tpu_sc as plsc

_N = 10000
_E = 320000
_D = 128
_H = 128
_DE = 16
_NLAYERS = 5
_NG = 256

_NC = 2
_NS = 16
_NW = _NC * _NS
_J = 40
_CH = 16
_KB0 = 352
_KB1 = 160
_NB = _NS * (_KB0 + _KB1)
_EP = _NB * _J
_AN = 10112
_RPT = _AN // _NS
_RCH = (128, 128, 128, 128, 120)


def _sc_edge_body(h_hbm, ea_hbm, src_hbm, dst_hbm, out_hbm,
                  is0, is1, id0, id1, g0, g1, e0, e1, s0, s1,
                  ism0, ism1, gs0, gs1, es0, es1, ss0, ss1, aggr):
    cid = lax.axis_index("c")
    sid = lax.axis_index("s")

    isb = (is0, is1)
    idb = (id0, id1)
    isem = (ism0, ism1)
    gbufs = (g0, g1)
    ebufs = (e0, e1)
    sbufs = (s0, s1)
    gsems = (gs0, gs1)
    esems = (es0, es1)
    ssems = (ss0, ss1)

    zero16 = jnp.zeros((16,), jnp.float32)

    def _zrow(i, c):
        for cc in range(_H // 16):
            g0[i, pl.ds(cc * 16, 16)] = zero16
        return c

    lax.fori_loop(0, _J, _zrow, 0)
    base = sid * _RPT
    for r in range(_RPT // _J):
        pltpu.sync_copy(g0, aggr.at[pl.ds(base + r * _J, _J)])
    rem = _RPT - (_RPT // _J) * _J
    if rem:
        pltpu.sync_copy(g0.at[pl.ds(0, rem)],
                        aggr.at[pl.ds(base + _RPT - rem, rem)])
    plsc.subcore_barrier()

    def _issue_ge(isrow, j, p):
        pltpu.async_copy(h_hbm.at[isrow], gbufs[p], gsems[p])
        pltpu.async_copy(ea_hbm.at[pl.ds(j * _J, _J)], ebufs[p], esems[p])

    def _wait_ge(isrow, j, p):
        pltpu.make_async_copy(h_hbm.at[isrow], gbufs[p], gsems[p]).wait()
        pltpu.make_async_copy(ea_hbm.at[pl.ds(j * _J, _J)], ebufs[p],
                              esems[p]).wait()

    def _compute(p):
        gb, eb, sb = gbufs[p], ebufs[p], sbufs[p]

        @plsc.parallel_loop(0, _J)
        def _(i):
            for cc in range(_H // 16):
                sl = pl.ds(cc * 16, 16)
                sb[i, sl] = jnp.maximum(gb[i, sl] + eb[i, sl], 0.0)

    def _issue_scatter(idrow, p):
        pltpu.async_copy(sbufs[p], aggr.at[idrow], ssems[p], add=True)

    def _wait_scatter(idrow, p):
        pltpu.make_async_copy(sbufs[p], aggr.at[idrow], ssems[p]).wait()

    bstart = jnp.where(cid == 0, sid * _KB0, _NS * _KB0 + sid * _KB1)
    npair = jnp.where(cid == 0, _KB0 // (2 * _CH), _KB1 // (2 * _CH))

    def _stage(cidx, par):
        sl = pl.ds(bstart + cidx * _CH, _CH)
        pltpu.async_copy(src_hbm.at[sl], isb[par], isem[par])
        pltpu.async_copy(dst_hbm.at[sl], idb[par], isem[par])

    def _wait_stage(cidx, par):
        sl = pl.ds(bstart + cidx * _CH, _CH)
        pltpu.make_async_copy(src_hbm.at[sl], isb[par], isem[par]).wait()
        pltpu.make_async_copy(dst_hbm.at[sl], idb[par], isem[par]).wait()

    def _process(cidx, par):
        isv, idv = isb[par], idb[par]
        jbase = bstart + cidx * _CH
        _issue_ge(isv.at[0], jbase + 0, 0)
        _issue_ge(isv.at[1], jbase + 1, 1)
        for b in (0, 1):
            _wait_ge(isv.at[b], jbase + b, b)
            _compute(b)
            _issue_scatter(idv.at[b], b)
            _issue_ge(isv.at[b + 2], jbase + b + 2, b)

        def _pair(q, c):
            for p in (0, 1):
                b = 2 * q + p
                _wait_ge(isv.at[b], jbase + b, p)
                _wait_scatter(idv.at[b - 2], p)
                _compute(p)
                _issue_scatter(idv.at[b], p)
                _issue_ge(isv.at[b + 2], jbase + b + 2, p)
            return c

        lax.fori_loop(1, _CH // 2 - 1, _pair, 0)
        for p in (0, 1):
            b = _CH - 2 + p
            _wait_ge(isv.at[b], jbase + b, p)
            _wait_scatter(idv.at[b - 2], p)
            _compute(p)
            _issue_scatter(idv.at[b], p)
        for p in (0, 1):
            _wait_scatter(idv.at[_CH - 2 + p], p)

    _stage(0, 0)

    def _qbody(q, c):
        _wait_stage(2 * q, 0)
        _stage(2 * q + 1, 1)
        _process(2 * q, 0)
        _wait_stage(2 * q + 1, 1)

        @pl.when(q < npair - 1)
        def _():
            _stage(2 * q + 2, 0)

        _process(2 * q + 1, 1)
        return c

    lax.fori_loop(0, npair, _qbody, 0)

    plsc.subcore_barrier()
    off = 0
    for rc in _RCH:
        sl = pl.ds(base + off, rc)
        pltpu.sync_copy(aggr.at[sl], out_hbm.at[cid, sl])
        off += rc


_sc_edge = pl.kernel(
    _sc_edge_body,
    out_type=jax.ShapeDtypeStruct((_NC, _AN, _H), jnp.float32),
    mesh=plsc.VectorSubcoreMesh(core_axis_name="c", subcore_axis_name="s",
                                num_cores=_NC, num_subcores=_NS),
    scratch_types=[
        pltpu.VMEM((_CH, _J), jnp.int32),
        pltpu.VMEM((_CH, _J), jnp.int32),
        pltpu.VMEM((_CH, _J), jnp.int32),
        pltpu.VMEM((_CH, _J), jnp.int32),
        pltpu.VMEM((_J, _H), jnp.float32),
        pltpu.VMEM((_J, _H), jnp.float32),
        pltpu.VMEM((_J, _H), jnp.float32),
        pltpu.VMEM((_J, _H), jnp.float32),
        pltpu.VMEM((_J, _H), jnp.float32),
        pltpu.VMEM((_J, _H), jnp.float32),
        pltpu.SemaphoreType.DMA,
        pltpu.SemaphoreType.DMA,
        pltpu.SemaphoreType.DMA,
        pltpu.SemaphoreType.DMA,
        pltpu.SemaphoreType.DMA,
        pltpu.SemaphoreType.DMA,
        pltpu.SemaphoreType.DMA,
        pltpu.SemaphoreType.DMA,
        pltpu.VMEM_SHARED((_AN, _H), jnp.float32),
    ],
)


def _ea_body(attr, wb, bb, out):
    out[...] = jnp.dot(attr[...], wb[...],
                       preferred_element_type=jnp.float32) + bb[...]


_BE = 4096
_ea_call = pl.pallas_call(
    _ea_body,
    grid=(_EP // _BE,),
    in_specs=[
        pl.BlockSpec((_BE, _DE), lambda i: (i, 0)),
        pl.BlockSpec((_DE, _H), lambda i: (0, 0)),
        pl.BlockSpec((1, _H), lambda i: (0, 0)),
    ],
    out_specs=pl.BlockSpec((_BE, _H), lambda i: (i, 0)),
    out_shape=jax.ShapeDtypeStruct((_EP, _H), jnp.float32),
)


def _dense_body(h, agg, w1, b1, g1, be1, w2, b2, g2, be2, wfc, s_in,
                h_out, s_out):
    z = h[...] + agg[0, :_N, :] + agg[1, :_N, :]
    a = jnp.dot(z, w1[...], preferred_element_type=jnp.float32) + b1[...]
    m = jnp.mean(a, axis=0, keepdims=True)
    v = jnp.mean((a - m) ** 2, axis=0, keepdims=True)
    a = jnp.maximum((a - m) / jnp.sqrt(v + 1e-5) * g1[...] + be1[...], 0.0)
    zz = jnp.dot(a, w2[...], preferred_element_type=jnp.float32) + b2[...]
    m2 = jnp.mean(zz, axis=0, keepdims=True)
    v2 = jnp.mean((zz - m2) ** 2, axis=0, keepdims=True)
    hh = jnp.maximum((zz - m2) / jnp.sqrt(v2 + 1e-5) * g2[...] + be2[...], 0.0)
    h_out[...] = hh
    s_out[...] = s_in[...] + jnp.dot(hh, wfc[...],
                                     preferred_element_type=jnp.float32)


_dense_call = pl.pallas_call(
    _dense_body,
    out_shape=(
        jax.ShapeDtypeStruct((_N, _H), jnp.float32),
        jax.ShapeDtypeStruct((_N, _H), jnp.float32),
    ),
)


def _pool_body(s, x, wfc0, bstack, batch, out):
    total = s[...] + jnp.dot(x[...], wfc0[...],
                             preferred_element_type=jnp.float32)
    gid = lax.broadcasted_iota(jnp.int32, (_NG, _N), 0)
    p = jnp.where(gid == batch[...], 1.0, 0.0)
    pooled = jnp.dot(p, total, preferred_element_type=jnp.float32)
    counts = jnp.sum(p, axis=1, keepdims=True)
    out[...] = pooled / jnp.maximum(counts, 1.0) + jnp.sum(
        bstack[...], axis=0, keepdims=True)


_pool_call = pl.pallas_call(
    _pool_body,
    out_shape=jax.ShapeDtypeStruct((_NG, _H), jnp.float32),
)


def kernel(x, edge_attr, params, edge_index, batch):
    src = edge_index[0].astype(jnp.int32)
    dst = edge_index[1].astype(jnp.int32)
    src = jnp.pad(src, (0, _EP - _E)).reshape(_NB, _J)
    dst = jnp.pad(dst, (0, _EP - _E), constant_values=_N).reshape(_NB, _J)
    ea_pad = jnp.pad(edge_attr.astype(jnp.float32), ((0, _EP - _E), (0, 0)))
    b2d = batch.astype(jnp.int32).reshape(1, _N)

    h = x.astype(jnp.float32)
    s = jnp.zeros((_N, _H), jnp.float32)
    for i in range(_NLAYERS):
        p = params["layers"][i]
        ea = _ea_call(ea_pad, p["Wb"], p["bb"].reshape(1, _H))
        agg = _sc_edge(h, ea, src, dst)
        h, s = _dense_call(
            h, agg, p["W1"], p["b1"].reshape(1, _H), p["g1"].reshape(1, _H),
            p["be1"].reshape(1, _H), p["W2"], p["b2"].reshape(1, _H),
            p["g_out"].reshape(1, _H), p["be_out"].reshape(1, _H),
            params["fcs"][i + 1][0], s)

    bstack = jnp.stack([b for (_w, b) in params["fcs"]])
    return _pool_call(s, x.astype(jnp.float32),
                      params["fcs"][0][0], bstack, b2d)
